# Initial kernel scaffold; baseline (speedup 1.0000x reference)
import jax
import jax.numpy as jnp
from jax.experimental import pallas as pl


def kernel(edges, node_fts, edge_fts, graph_fts, adj, W_node, W_edge, a_src, a_dst, a_edge, W_ih, W_hh, b_ih, b_hh, W_ffn, b_ffn):
    raise NotImplementedError("write your pallas kernel here")


# speedup vs baseline: 60.4889x; 60.4889x over previous
import functools

import jax
import jax.numpy as jnp
from jax import lax
from jax.experimental import pallas as pl
from jax.experimental.pallas import tpu as pltpu


def _leaky(x):
    return jnp.where(x >= 0, x, 0.2 * x)


def _gn_block_kernel(nf_ref, ef_ref, src_col_ref, dst_col_ref, dst_row_ref,
                     wn_ref, we_ref, asrc_ref, adst_ref, aedge_ref, out_ref):
    N = nf_ref.shape[1]
    E = ef_ref.shape[1]
    H = wn_ref.shape[0]
    NO = wn_ref.shape[2]

    nf = nf_ref[0]
    ef = ef_ref[0]
    src_col = src_col_ref[0]
    dst_col = dst_col_ref[0]
    dst_row = dst_row_ref[0]

    iota_row = lax.broadcasted_iota(jnp.int32, (E, N), 1)
    oh_src = (src_col == iota_row).astype(jnp.float32)
    oh_dst = (dst_col == iota_row).astype(jnp.float32)
    iota_col = lax.broadcasted_iota(jnp.int32, (N, E), 0)
    oh_dstT = (iota_col == dst_row).astype(jnp.float32)

    h_b = []
    s_src_rows = []
    s_dst_rows = []
    w_e_rows = []
    for h in range(H):
        hb = jnp.dot(nf, wn_ref[h], preferred_element_type=jnp.float32)
        h_b.append(hb)
        s_src_rows.append(lax.dot_general(
            asrc_ref[h:h + 1, :], hb, (((1,), (1,)), ((), ())),
            preferred_element_type=jnp.float32))
        s_dst_rows.append(lax.dot_general(
            adst_ref[h:h + 1, :], hb, (((1,), (1,)), ((), ())),
            preferred_element_type=jnp.float32))
        w_e_rows.append(lax.dot_general(
            aedge_ref[h:h + 1, :], we_ref[h], (((1,), (1,)), ((), ())),
            preferred_element_type=jnp.float32))
    s_src_mat = jnp.concatenate(s_src_rows, axis=0)
    s_dst_mat = jnp.concatenate(s_dst_rows, axis=0)
    w_e_mat = jnp.concatenate(w_e_rows, axis=0)

    l_src = lax.dot_general(s_src_mat, oh_src, (((1,), (1,)), ((), ())),
                            preferred_element_type=jnp.float32)
    l_dst = lax.dot_general(s_dst_mat, oh_dst, (((1,), (1,)), ((), ())),
                            preferred_element_type=jnp.float32)
    s_e = lax.dot_general(w_e_mat, ef, (((1,), (1,)), ((), ())),
                          preferred_element_type=jnp.float32)
    logits = _leaky(l_src + l_dst + s_e)

    m = jnp.max(logits, axis=1, keepdims=True)
    ex = jnp.exp(logits - m)
    den = lax.dot_general(ex, oh_dst, (((1,), (0,)), ((), ())),
                          preferred_element_type=jnp.float32)
    den_e = lax.dot_general(den, oh_dst, (((1,), (1,)), ((), ())),
                            preferred_element_type=jnp.float32)
    alpha = ex / (den_e + 1e-16)

    acc = jnp.zeros((N, NO), jnp.float32)
    for h in range(H):
        h_src = jnp.dot(oh_src, h_b[h], preferred_element_type=jnp.float32)
        ohw = oh_dstT * alpha[h:h + 1, :]
        acc = acc + jnp.dot(ohw, h_src, preferred_element_type=jnp.float32)
    out_ref[0] = _leaky(acc * (1.0 / H))


def _gi_kernel(xs_ref, wih_ref, bih_ref, out_ref):
    k = pl.program_id(0)

    @pl.when(k == 0)
    def _():
        out_ref[...] = jnp.broadcast_to(bih_ref[...], out_ref.shape)

    out_ref[...] += lax.dot_general(
        xs_ref[...], wih_ref[...], (((1,), (1,)), ((), ())),
        preferred_element_type=jnp.float32)


def _gru_kernel(gi_ref, whh_ref, bhh_ref, out_ref):
    T = gi_ref.shape[0]
    GH = whh_ref.shape[1]

    def step(t, hprev):
        gi_t = gi_ref[pl.ds(t, 1), :]
        gh = lax.dot_general(hprev, whh_ref[...], (((1,), (1,)), ((), ())),
                             preferred_element_type=jnp.float32) + bhh_ref[...]
        i_r = gi_t[:, 0:GH]
        i_z = gi_t[:, GH:2 * GH]
        i_n = gi_t[:, 2 * GH:3 * GH]
        h_r = gh[:, 0:GH]
        h_z = gh[:, GH:2 * GH]
        h_n = gh[:, 2 * GH:3 * GH]
        r = jax.nn.sigmoid(i_r + h_r)
        z = jax.nn.sigmoid(i_z + h_z)
        n = jnp.tanh(i_n + r * h_n)
        return (1.0 - z) * n + z * hprev

    out_ref[...] = lax.fori_loop(jnp.int32(0), jnp.int32(T), step,
                                 jnp.zeros((1, GH), jnp.float32))


def _ffn_kernel(h_ref, wffn_ref, bffn_ref, out_ref):
    out_ref[...] = jax.nn.sigmoid(
        jnp.dot(h_ref[...], wffn_ref[...], preferred_element_type=jnp.float32)
        + bffn_ref[...])


def kernel(edges, node_fts, edge_fts, graph_fts, adj, W_node, W_edge, a_src,
           a_dst, a_edge, W_ih, W_hh, b_ih, b_hh, W_ffn, b_ffn):
    with jax.enable_x64(False):
        return _kernel_impl(edges, node_fts, edge_fts, graph_fts, adj, W_node,
                            W_edge, a_src, a_dst, a_edge, W_ih, W_hh, b_ih,
                            b_hh, W_ffn, b_ffn)


def _kernel_impl(edges, node_fts, edge_fts, graph_fts, adj, W_node, W_edge,
                 a_src, a_dst, a_edge, W_ih, W_hh, b_ih, b_hh, W_ffn, b_ffn):
    T, N, NODE_IN = node_fts.shape
    E = edge_fts.shape[1]
    EDGE_IN = edge_fts.shape[2]
    H, _, NO = W_node.shape
    GH = W_hh.shape[1]
    V = W_ffn.shape[1]

    src = edges[:, 0, :].astype(jnp.int32)
    dst = edges[:, 1, :].astype(jnp.int32)
    src_col = src.reshape(T, E, 1)
    dst_col = dst.reshape(T, E, 1)
    dst_row = dst.reshape(T, 1, E)

    xs = pl.pallas_call(
        _gn_block_kernel,
        name='gn_block',
        grid=(T,),
        in_specs=[
            pl.BlockSpec((1, N, NODE_IN), lambda t: (t, 0, 0)),
            pl.BlockSpec((1, E, EDGE_IN), lambda t: (t, 0, 0)),
            pl.BlockSpec((1, E, 1), lambda t: (t, 0, 0)),
            pl.BlockSpec((1, E, 1), lambda t: (t, 0, 0)),
            pl.BlockSpec((1, 1, E), lambda t: (t, 0, 0)),
            pl.BlockSpec((H, NODE_IN, NO), lambda t: (0, 0, 0)),
            pl.BlockSpec(W_edge.shape, lambda t: (0, 0, 0)),
            pl.BlockSpec(a_src.shape, lambda t: (0, 0)),
            pl.BlockSpec(a_dst.shape, lambda t: (0, 0)),
            pl.BlockSpec(a_edge.shape, lambda t: (0, 0)),
        ],
        out_specs=pl.BlockSpec((1, N, NO), lambda t: (t, 0, 0)),
        out_shape=jax.ShapeDtypeStruct((T, N, NO), jnp.float32),
    )(node_fts, edge_fts, src_col, dst_col, dst_row,
      W_node, W_edge, a_src, a_dst, a_edge)

    xs2 = xs.reshape(T, N * NO)

    K = N * NO
    KC = 2048
    gi = pl.pallas_call(
        _gi_kernel,
        name='gi',
        grid=(K // KC,),
        in_specs=[
            pl.BlockSpec((T, KC), lambda k: (0, k)),
            pl.BlockSpec((3 * GH, KC), lambda k: (0, k)),
            pl.BlockSpec((1, 3 * GH), lambda k: (0, 0)),
        ],
        out_specs=pl.BlockSpec((T, 3 * GH), lambda k: (0, 0)),
        out_shape=jax.ShapeDtypeStruct((T, 3 * GH), jnp.float32),
    )(xs2, W_ih, b_ih.reshape(1, 3 * GH))

    hT = pl.pallas_call(
        _gru_kernel,
        name='gru',
        in_specs=[
            pl.BlockSpec((T, 3 * GH), lambda: (0, 0)),
            pl.BlockSpec((3 * GH, GH), lambda: (0, 0)),
            pl.BlockSpec((1, 3 * GH), lambda: (0, 0)),
        ],
        out_specs=pl.BlockSpec((1, GH), lambda: (0, 0)),
        out_shape=jax.ShapeDtypeStruct((1, GH), jnp.float32),
    )(gi, W_hh, b_hh.reshape(1, 3 * GH))

    VC = 4096
    out = pl.pallas_call(
        _ffn_kernel,
        name='ffn',
        grid=(V // VC,),
        in_specs=[
            pl.BlockSpec((1, GH), lambda v: (0, 0)),
            pl.BlockSpec((GH, VC), lambda v: (0, v)),
            pl.BlockSpec((1, VC), lambda v: (0, v)),
        ],
        out_specs=pl.BlockSpec((1, VC), lambda v: (0, v)),
        out_shape=jax.ShapeDtypeStruct((1, V), jnp.float32),
    )(hT, W_ffn, b_ffn.reshape(1, V))

    return out


# speedup vs baseline: 125.7541x; 2.0790x over previous
import functools

import jax
import jax.numpy as jnp
from jax import lax
from jax.experimental import pallas as pl
from jax.experimental.pallas import tpu as pltpu
from jax.experimental.pallas import tpu_sc as plsc

_NC = 2
_NS = 16
_LANES = 16


def _leaky(x):
    return jnp.where(x >= 0, x, 0.2 * x)


def _pre_kernel(nf_ref, ef_ref, wn_ref, we_ref, asrc_ref, adst_ref,
                aedge_ref, h_ref, ssrc_ref, sdst_ref, se_ref):
    H = wn_ref.shape[0]
    nf = nf_ref[0]
    ef = ef_ref[0]
    for h in range(H):
        hb = jnp.dot(nf, wn_ref[h], preferred_element_type=jnp.float32)
        h_ref[0, h] = hb
        ssrc_ref[0, h:h + 1, :] = lax.dot_general(
            asrc_ref[h:h + 1, :], hb, (((1,), (1,)), ((), ())),
            preferred_element_type=jnp.float32)
        sdst_ref[0, h:h + 1, :] = lax.dot_general(
            adst_ref[h:h + 1, :], hb, (((1,), (1,)), ((), ())),
            preferred_element_type=jnp.float32)
        w_e = lax.dot_general(
            aedge_ref[h:h + 1, :], we_ref[h], (((1,), (1,)), ((), ())),
            preferred_element_type=jnp.float32)
        se_ref[0, h:h + 1, :] = lax.dot_general(
            w_e, ef, (((1,), (1,)), ((), ())),
            preferred_element_type=jnp.float32)


def _make_sc_edge_kernel(T, H, N, E):
    TASKS_PER_W = (T * H) // (_NC * _NS)
    NCHUNK = E // _LANES
    mesh = plsc.VectorSubcoreMesh(core_axis_name="c", subcore_axis_name="s",
                                  num_cores=_NC, num_subcores=_NS)

    @functools.partial(
        pl.kernel,
        mesh=mesh,
        compiler_params=pltpu.CompilerParams(needs_layout_passes=False),
        out_type=(
            jax.ShapeDtypeStruct((T, H, N, N), jnp.float32),
            jax.ShapeDtypeStruct((T, H, N), jnp.float32),
        ),
        scratch_types=[
            pltpu.VMEM((E,), jnp.int32),
            pltpu.VMEM((E,), jnp.int32),
            pltpu.VMEM((N,), jnp.float32),
            pltpu.VMEM((N,), jnp.float32),
            pltpu.VMEM((E,), jnp.float32),
            pltpu.VMEM((E,), jnp.float32),
            pltpu.VMEM((N, N), jnp.float32),
            pltpu.VMEM((N,), jnp.float32),
        ],
    )
    def sc_edge(src_hbm, dst_hbm, ssrc_hbm, sdst_hbm, se_hbm, zeros_hbm,
                a_out, den_out, src_v, dst_v, ssrc_v, sdst_v, se_v, l_v,
                a_v, den_v):
        wid = lax.axis_index("s") * _NC + lax.axis_index("c")
        base = wid * TASKS_PER_W
        for k in range(TASKS_PER_W):
            task = base + k
            t = task // H
            h = k % H
            if h == 0:
                pltpu.sync_copy(src_hbm.at[t], src_v)
                pltpu.sync_copy(dst_hbm.at[t], dst_v)
            pltpu.sync_copy(ssrc_hbm.at[t, h], ssrc_v)
            pltpu.sync_copy(sdst_hbm.at[t, h], sdst_v)
            pltpu.sync_copy(se_hbm.at[t, h], se_v)
            pltpu.sync_copy(zeros_hbm, a_v)
            pltpu.sync_copy(zeros_hbm.at[0], den_v)

            def pass1(i, m):
                sl = pl.ds(i * _LANES, _LANES)
                v = (plsc.load_gather(ssrc_v, [src_v[sl]])
                     + plsc.load_gather(sdst_v, [dst_v[sl]])
                     + se_v[sl])
                lg = jnp.where(v >= 0, v, 0.2 * v)
                l_v[sl] = lg
                return jnp.maximum(m, lg)

            m16 = lax.fori_loop(jnp.int32(0), jnp.int32(NCHUNK), pass1,
                                jnp.full((_LANES,), -jnp.inf, jnp.float32))
            mx = jnp.max(m16)

            def pass2(i, carry):
                sl = pl.ds(i * _LANES, _LANES)
                ex = jnp.exp(l_v[sl] - mx)
                d_idx = dst_v[sl]
                plsc.addupdate_scatter(den_v, [d_idx], ex)
                plsc.addupdate_scatter(a_v, [d_idx, src_v[sl]], ex)
                return carry

            lax.fori_loop(jnp.int32(0), jnp.int32(NCHUNK), pass2,
                          jnp.int32(0))
            pltpu.sync_copy(a_v, a_out.at[t, h])
            pltpu.sync_copy(den_v, den_out.at[t, h])

    return sc_edge


def _agg_kernel(a_ref, h_ref, den_ref, out_ref):
    H = h_ref.shape[1]
    N, NO = h_ref.shape[2], h_ref.shape[3]
    acc = jnp.zeros((N, NO), jnp.float32)
    for h in range(H):
        agg = jnp.dot(a_ref[0, h], h_ref[0, h],
                      preferred_element_type=jnp.float32)
        rec = 1.0 / (den_ref[0, h] + 1e-16)
        acc = acc + agg * rec
    out_ref[0] = _leaky(acc * (1.0 / H))


def _gi_kernel(xs_ref, wih_ref, bih_ref, out_ref):
    k = pl.program_id(0)

    @pl.when(k == 0)
    def _():
        out_ref[...] = jnp.broadcast_to(bih_ref[...], out_ref.shape)

    out_ref[...] += lax.dot_general(
        xs_ref[...], wih_ref[...], (((1,), (1,)), ((), ())),
        preferred_element_type=jnp.float32)


def _gru_kernel(gi_ref, whh_ref, bhh_ref, out_ref):
    T = gi_ref.shape[0]
    GH = whh_ref.shape[1]

    def step(t, hprev):
        gi_t = gi_ref[pl.ds(t, 1), :]
        gh = lax.dot_general(hprev, whh_ref[...], (((1,), (1,)), ((), ())),
                             preferred_element_type=jnp.float32) + bhh_ref[...]
        i_r = gi_t[:, 0:GH]
        i_z = gi_t[:, GH:2 * GH]
        i_n = gi_t[:, 2 * GH:3 * GH]
        h_r = gh[:, 0:GH]
        h_z = gh[:, GH:2 * GH]
        h_n = gh[:, 2 * GH:3 * GH]
        r = jax.nn.sigmoid(i_r + h_r)
        z = jax.nn.sigmoid(i_z + h_z)
        n = jnp.tanh(i_n + r * h_n)
        return (1.0 - z) * n + z * hprev

    out_ref[...] = lax.fori_loop(jnp.int32(0), jnp.int32(T), step,
                                 jnp.zeros((1, GH), jnp.float32))


def _ffn_kernel(h_ref, wffn_ref, bffn_ref, out_ref):
    out_ref[...] = jax.nn.sigmoid(
        jnp.dot(h_ref[...], wffn_ref[...], preferred_element_type=jnp.float32)
        + bffn_ref[...])


def kernel(edges, node_fts, edge_fts, graph_fts, adj, W_node, W_edge, a_src,
           a_dst, a_edge, W_ih, W_hh, b_ih, b_hh, W_ffn, b_ffn):
    with jax.enable_x64(False):
        return _kernel_impl(edges, node_fts, edge_fts, graph_fts, adj, W_node,
                            W_edge, a_src, a_dst, a_edge, W_ih, W_hh, b_ih,
                            b_hh, W_ffn, b_ffn)


def _kernel_impl(edges, node_fts, edge_fts, graph_fts, adj, W_node, W_edge,
                 a_src, a_dst, a_edge, W_ih, W_hh, b_ih, b_hh, W_ffn, b_ffn):
    T, N, NODE_IN = node_fts.shape
    E = edge_fts.shape[1]
    EDGE_IN = edge_fts.shape[2]
    H, _, NO = W_node.shape
    GH = W_hh.shape[1]
    V = W_ffn.shape[1]

    src = edges[:, 0, :].astype(jnp.int32)
    dst = edges[:, 1, :].astype(jnp.int32)

    h_all, s_src, s_dst, s_e = pl.pallas_call(
        _pre_kernel,
        name='gn_pre',
        grid=(T,),
        in_specs=[
            pl.BlockSpec((1, N, NODE_IN), lambda t: (t, 0, 0)),
            pl.BlockSpec((1, E, EDGE_IN), lambda t: (t, 0, 0)),
            pl.BlockSpec((H, NODE_IN, NO), lambda t: (0, 0, 0)),
            pl.BlockSpec(W_edge.shape, lambda t: (0, 0, 0)),
            pl.BlockSpec(a_src.shape, lambda t: (0, 0)),
            pl.BlockSpec(a_dst.shape, lambda t: (0, 0)),
            pl.BlockSpec(a_edge.shape, lambda t: (0, 0)),
        ],
        out_specs=[
            pl.BlockSpec((1, H, N, NO), lambda t: (t, 0, 0, 0)),
            pl.BlockSpec((1, H, N), lambda t: (t, 0, 0)),
            pl.BlockSpec((1, H, N), lambda t: (t, 0, 0)),
            pl.BlockSpec((1, H, E), lambda t: (t, 0, 0)),
        ],
        out_shape=[
            jax.ShapeDtypeStruct((T, H, N, NO), jnp.float32),
            jax.ShapeDtypeStruct((T, H, N), jnp.float32),
            jax.ShapeDtypeStruct((T, H, N), jnp.float32),
            jax.ShapeDtypeStruct((T, H, E), jnp.float32),
        ],
    )(node_fts, edge_fts, W_node, W_edge, a_src, a_dst, a_edge)

    zeros2d = jnp.zeros((N, N), jnp.float32)
    sc_edge = _make_sc_edge_kernel(T, H, N, E)
    a_mat, den = sc_edge(src, dst, s_src, s_dst, s_e, zeros2d)

    den_col = den.reshape(T, H, N, 1)
    xs = pl.pallas_call(
        _agg_kernel,
        name='gn_agg',
        grid=(T,),
        in_specs=[
            pl.BlockSpec((1, H, N, N), lambda t: (t, 0, 0, 0)),
            pl.BlockSpec((1, H, N, NO), lambda t: (t, 0, 0, 0)),
            pl.BlockSpec((1, H, N, 1), lambda t: (t, 0, 0, 0)),
        ],
        out_specs=pl.BlockSpec((1, N, NO), lambda t: (t, 0, 0)),
        out_shape=jax.ShapeDtypeStruct((T, N, NO), jnp.float32),
    )(a_mat, h_all, den_col)

    xs2 = xs.reshape(T, N * NO)

    K = N * NO
    KC = 2048
    gi = pl.pallas_call(
        _gi_kernel,
        name='gi',
        grid=(K // KC,),
        in_specs=[
            pl.BlockSpec((T, KC), lambda k: (0, k)),
            pl.BlockSpec((3 * GH, KC), lambda k: (0, k)),
            pl.BlockSpec((1, 3 * GH), lambda k: (0, 0)),
        ],
        out_specs=pl.BlockSpec((T, 3 * GH), lambda k: (0, 0)),
        out_shape=jax.ShapeDtypeStruct((T, 3 * GH), jnp.float32),
    )(xs2, W_ih, b_ih.reshape(1, 3 * GH))

    hT = pl.pallas_call(
        _gru_kernel,
        name='gru',
        in_specs=[
            pl.BlockSpec((T, 3 * GH), lambda: (0, 0)),
            pl.BlockSpec((3 * GH, GH), lambda: (0, 0)),
            pl.BlockSpec((1, 3 * GH), lambda: (0, 0)),
        ],
        out_specs=pl.BlockSpec((1, GH), lambda: (0, 0)),
        out_shape=jax.ShapeDtypeStruct((1, GH), jnp.float32),
    )(gi, W_hh, b_hh.reshape(1, 3 * GH))

    VC = 4096
    out = pl.pallas_call(
        _ffn_kernel,
        name='ffn',
        grid=(V // VC,),
        in_specs=[
            pl.BlockSpec((1, GH), lambda v: (0, 0)),
            pl.BlockSpec((GH, VC), lambda v: (0, v)),
            pl.BlockSpec((1, VC), lambda v: (0, v)),
        ],
        out_specs=pl.BlockSpec((1, VC), lambda v: (0, v)),
        out_shape=jax.ShapeDtypeStruct((1, V), jnp.float32),
    )(hT, W_ffn, b_ffn.reshape(1, V))

    return out


# speedup vs baseline: 162.9759x; 1.2960x over previous
import functools

import jax
import jax.numpy as jnp
from jax import lax
from jax.experimental import pallas as pl
from jax.experimental.pallas import tpu as pltpu
from jax.experimental.pallas import tpu_sc as plsc

_NC = 2
_NS = 16
_LANES = 16


def _leaky(x):
    return jnp.where(x >= 0, x, 0.2 * x)


def _pre_kernel(nf_ref, ef_ref, wn_ref, we_ref, asrc_ref, adst_ref,
                aedge_ref, h_ref, ssrc_ref, sdst_ref, se_ref):
    H = wn_ref.shape[0]
    NO = wn_ref.shape[2]
    nf = nf_ref[0]
    ef = ef_ref[0]
    wn_cat = jnp.concatenate([wn_ref[h] for h in range(H)], axis=1)
    hb_all = jnp.dot(nf, wn_cat, preferred_element_type=jnp.float32)
    wsrc_rows = []
    wdst_rows = []
    we_rows = []
    for h in range(H):
        h_ref[0, h] = hb_all[:, h * NO:(h + 1) * NO]
        wsrc_rows.append(lax.dot_general(
            asrc_ref[h:h + 1, :], wn_ref[h], (((1,), (1,)), ((), ())),
            preferred_element_type=jnp.float32))
        wdst_rows.append(lax.dot_general(
            adst_ref[h:h + 1, :], wn_ref[h], (((1,), (1,)), ((), ())),
            preferred_element_type=jnp.float32))
        we_rows.append(lax.dot_general(
            aedge_ref[h:h + 1, :], we_ref[h], (((1,), (1,)), ((), ())),
            preferred_element_type=jnp.float32))
    wsrc = jnp.concatenate(wsrc_rows, axis=0)
    wdst = jnp.concatenate(wdst_rows, axis=0)
    wem = jnp.concatenate(we_rows, axis=0)
    ssrc_ref[0] = lax.dot_general(wsrc, nf, (((1,), (1,)), ((), ())),
                                  preferred_element_type=jnp.float32)
    sdst_ref[0] = lax.dot_general(wdst, nf, (((1,), (1,)), ((), ())),
                                  preferred_element_type=jnp.float32)
    se_ref[0] = lax.dot_general(wem, ef, (((1,), (1,)), ((), ())),
                                preferred_element_type=jnp.float32)


def _make_sc_edge_kernel(T, H, N, E):
    TASKS_PER_W = (T * H) // (_NC * _NS)
    NCHUNK = E // _LANES
    mesh = plsc.VectorSubcoreMesh(core_axis_name="c", subcore_axis_name="s",
                                  num_cores=_NC, num_subcores=_NS)

    @functools.partial(
        pl.kernel,
        mesh=mesh,
        compiler_params=pltpu.CompilerParams(needs_layout_passes=False),
        out_type=jax.ShapeDtypeStruct((T, H, N, N), jnp.float32),
        scratch_types=[
            pltpu.VMEM((E,), jnp.int32),
            pltpu.VMEM((E,), jnp.int32),
            pltpu.VMEM((N,), jnp.float32),
            pltpu.VMEM((N,), jnp.float32),
            pltpu.VMEM((E,), jnp.float32),
            pltpu.VMEM((E,), jnp.float32),
            pltpu.VMEM((N, N), jnp.float32),
        ],
    )
    def sc_edge(src_hbm, dst_hbm, ssrc_hbm, sdst_hbm, se_hbm,
                a_out, src_v, dst_v, ssrc_v, sdst_v, se_v, l_v, a_v):
        wid = lax.axis_index("s") * _NC + lax.axis_index("c")
        base = wid * TASKS_PER_W
        z16 = jnp.zeros((_LANES,), jnp.float32)

        def zrow(r, carry):
            for cc in range(N // _LANES):
                a_v[r, pl.ds(cc * _LANES, _LANES)] = z16
            return carry

        lax.fori_loop(jnp.int32(0), jnp.int32(N), zrow, jnp.int32(0))

        for k in range(TASKS_PER_W):
            task = base + k
            t = task // H
            h = k % H
            if h == 0:
                pltpu.sync_copy(src_hbm.at[t], src_v)
                pltpu.sync_copy(dst_hbm.at[t], dst_v)
            pltpu.sync_copy(ssrc_hbm.at[t, h], ssrc_v)
            pltpu.sync_copy(sdst_hbm.at[t, h], sdst_v)
            pltpu.sync_copy(se_hbm.at[t, h], se_v)

            def pass1(i, m):
                sl = pl.ds(i * _LANES, _LANES)
                v = (plsc.load_gather(ssrc_v, [src_v[sl]])
                     + plsc.load_gather(sdst_v, [dst_v[sl]])
                     + se_v[sl])
                lg = jnp.where(v >= 0, v, 0.2 * v)
                l_v[sl] = lg
                return jnp.maximum(m, lg)

            m16 = lax.fori_loop(jnp.int32(0), jnp.int32(NCHUNK), pass1,
                                jnp.full((_LANES,), -jnp.inf, jnp.float32))
            mx = jnp.max(m16)

            def pass2(i, carry):
                sl = pl.ds(i * _LANES, _LANES)
                ex = jnp.exp(l_v[sl] - mx)
                plsc.addupdate_scatter(a_v, [dst_v[sl], src_v[sl]], ex)
                return carry

            lax.fori_loop(jnp.int32(0), jnp.int32(NCHUNK), pass2,
                          jnp.int32(0))
            pltpu.sync_copy(a_v, a_out.at[t, h])

            def pass3(i, carry):
                sl = pl.ds(i * _LANES, _LANES)
                plsc.store_scatter(a_v, [dst_v[sl], src_v[sl]], z16)
                return carry

            lax.fori_loop(jnp.int32(0), jnp.int32(NCHUNK), pass3,
                          jnp.int32(0))

    return sc_edge


def _agg_kernel(a_ref, h_ref, out_ref):
    H = h_ref.shape[1]
    N, NO = h_ref.shape[2], h_ref.shape[3]
    ones_col = jnp.ones((N, 1), jnp.float32)
    acc = jnp.zeros((N, NO), jnp.float32)
    for h in range(H):
        a = a_ref[0, h]
        agg = jnp.dot(a, h_ref[0, h],
                      preferred_element_type=jnp.float32)
        den = jnp.dot(a, ones_col, preferred_element_type=jnp.float32)
        rec = 1.0 / (den + 1e-16)
        acc = acc + agg * rec
    out_ref[0] = _leaky(acc * (1.0 / H))


def _gi_kernel(xs_ref, wih_ref, bih_ref, out_ref):
    k = pl.program_id(0)

    @pl.when(k == 0)
    def _():
        out_ref[...] = jnp.broadcast_to(bih_ref[...], out_ref.shape)

    out_ref[...] += lax.dot_general(
        xs_ref[...], wih_ref[...], (((1,), (1,)), ((), ())),
        preferred_element_type=jnp.float32)


def _gru_kernel(gi_ref, whh_ref, bhh_ref, out_ref):
    T = gi_ref.shape[0]
    GH = whh_ref.shape[1]

    def step(t, hprev):
        gi_t = gi_ref[pl.ds(t, 1), :]
        gh = lax.dot_general(hprev, whh_ref[...], (((1,), (1,)), ((), ())),
                             preferred_element_type=jnp.float32) + bhh_ref[...]
        i_r = gi_t[:, 0:GH]
        i_z = gi_t[:, GH:2 * GH]
        i_n = gi_t[:, 2 * GH:3 * GH]
        h_r = gh[:, 0:GH]
        h_z = gh[:, GH:2 * GH]
        h_n = gh[:, 2 * GH:3 * GH]
        r = jax.nn.sigmoid(i_r + h_r)
        z = jax.nn.sigmoid(i_z + h_z)
        n = jnp.tanh(i_n + r * h_n)
        return (1.0 - z) * n + z * hprev

    out_ref[...] = lax.fori_loop(jnp.int32(0), jnp.int32(T), step,
                                 jnp.zeros((1, GH), jnp.float32))


def _ffn_kernel(h_ref, wffn_ref, bffn_ref, out_ref):
    out_ref[...] = jax.nn.sigmoid(
        jnp.dot(h_ref[...], wffn_ref[...], preferred_element_type=jnp.float32)
        + bffn_ref[...])


def kernel(edges, node_fts, edge_fts, graph_fts, adj, W_node, W_edge, a_src,
           a_dst, a_edge, W_ih, W_hh, b_ih, b_hh, W_ffn, b_ffn):
    with jax.enable_x64(False):
        return _kernel_impl(edges, node_fts, edge_fts, graph_fts, adj, W_node,
                            W_edge, a_src, a_dst, a_edge, W_ih, W_hh, b_ih,
                            b_hh, W_ffn, b_ffn)


def _kernel_impl(edges, node_fts, edge_fts, graph_fts, adj, W_node, W_edge,
                 a_src, a_dst, a_edge, W_ih, W_hh, b_ih, b_hh, W_ffn, b_ffn):
    T, N, NODE_IN = node_fts.shape
    E = edge_fts.shape[1]
    EDGE_IN = edge_fts.shape[2]
    H, _, NO = W_node.shape
    GH = W_hh.shape[1]
    V = W_ffn.shape[1]

    src = edges[:, 0, :].astype(jnp.int32)
    dst = edges[:, 1, :].astype(jnp.int32)

    h_all, s_src, s_dst, s_e = pl.pallas_call(
        _pre_kernel,
        name='gn_pre',
        grid=(T,),
        in_specs=[
            pl.BlockSpec((1, N, NODE_IN), lambda t: (t, 0, 0)),
            pl.BlockSpec((1, E, EDGE_IN), lambda t: (t, 0, 0)),
            pl.BlockSpec((H, NODE_IN, NO), lambda t: (0, 0, 0)),
            pl.BlockSpec(W_edge.shape, lambda t: (0, 0, 0)),
            pl.BlockSpec(a_src.shape, lambda t: (0, 0)),
            pl.BlockSpec(a_dst.shape, lambda t: (0, 0)),
            pl.BlockSpec(a_edge.shape, lambda t: (0, 0)),
        ],
        out_specs=[
            pl.BlockSpec((1, H, N, NO), lambda t: (t, 0, 0, 0)),
            pl.BlockSpec((1, H, N), lambda t: (t, 0, 0)),
            pl.BlockSpec((1, H, N), lambda t: (t, 0, 0)),
            pl.BlockSpec((1, H, E), lambda t: (t, 0, 0)),
        ],
        out_shape=[
            jax.ShapeDtypeStruct((T, H, N, NO), jnp.float32),
            jax.ShapeDtypeStruct((T, H, N), jnp.float32),
            jax.ShapeDtypeStruct((T, H, N), jnp.float32),
            jax.ShapeDtypeStruct((T, H, E), jnp.float32),
        ],
    )(node_fts, edge_fts, W_node, W_edge, a_src, a_dst, a_edge)

    sc_edge = _make_sc_edge_kernel(T, H, N, E)
    a_mat = sc_edge(src, dst, s_src, s_dst, s_e)

    xs = pl.pallas_call(
        _agg_kernel,
        name='gn_agg',
        grid=(T,),
        in_specs=[
            pl.BlockSpec((1, H, N, N), lambda t: (t, 0, 0, 0)),
            pl.BlockSpec((1, H, N, NO), lambda t: (t, 0, 0, 0)),
        ],
        out_specs=pl.BlockSpec((1, N, NO), lambda t: (t, 0, 0)),
        out_shape=jax.ShapeDtypeStruct((T, N, NO), jnp.float32),
    )(a_mat, h_all)

    xs2 = xs.reshape(T, N * NO)

    K = N * NO
    KC = 2048
    gi = pl.pallas_call(
        _gi_kernel,
        name='gi',
        grid=(K // KC,),
        in_specs=[
            pl.BlockSpec((T, KC), lambda k: (0, k)),
            pl.BlockSpec((3 * GH, KC), lambda k: (0, k)),
            pl.BlockSpec((1, 3 * GH), lambda k: (0, 0)),
        ],
        out_specs=pl.BlockSpec((T, 3 * GH), lambda k: (0, 0)),
        out_shape=jax.ShapeDtypeStruct((T, 3 * GH), jnp.float32),
    )(xs2, W_ih, b_ih.reshape(1, 3 * GH))

    hT = pl.pallas_call(
        _gru_kernel,
        name='gru',
        in_specs=[
            pl.BlockSpec((T, 3 * GH), lambda: (0, 0)),
            pl.BlockSpec((3 * GH, GH), lambda: (0, 0)),
            pl.BlockSpec((1, 3 * GH), lambda: (0, 0)),
        ],
        out_specs=pl.BlockSpec((1, GH), lambda: (0, 0)),
        out_shape=jax.ShapeDtypeStruct((1, GH), jnp.float32),
    )(gi, W_hh, b_hh.reshape(1, 3 * GH))

    VC = 4096
    out = pl.pallas_call(
        _ffn_kernel,
        name='ffn',
        grid=(V // VC,),
        in_specs=[
            pl.BlockSpec((1, GH), lambda v: (0, 0)),
            pl.BlockSpec((GH, VC), lambda v: (0, v)),
            pl.BlockSpec((1, VC), lambda v: (0, v)),
        ],
        out_specs=pl.BlockSpec((1, VC), lambda v: (0, v)),
        out_shape=jax.ShapeDtypeStruct((1, V), jnp.float32),
    )(hT, W_ffn, b_ffn.reshape(1, V))

    return out
